# Initial kernel scaffold; baseline (speedup 1.0000x reference)
import jax
import jax.numpy as jnp
from jax.experimental import pallas as pl


def kernel(seq_len, line_emb, trigram_emb, hexagram_emb, seq_emb, scale):
    raise NotImplementedError("write your pallas kernel here")


# speedup vs baseline: 5.5817x; 5.5817x over previous
import jax
import jax.numpy as jnp
from jax.experimental import pallas as pl
from jax.experimental.pallas import tpu as pltpu

_D = 2048
_N = 8192
_BLK = 512


def _tc_body(scale_ref, line_ref, tri_ref, hex_ref, seq_ref, out_ref):
    i = pl.program_id(0)
    pos = i * _BLK + jax.lax.broadcasted_iota(jnp.int32, (_BLK, 1), 0)
    oh6 = (pos % 6 == jax.lax.broadcasted_iota(jnp.int32, (_BLK, 6), 1)
           ).astype(jnp.float32)
    oh2 = ((pos % 6) // 3 == jax.lax.broadcasted_iota(jnp.int32, (_BLK, 2), 1)
           ).astype(jnp.float32)
    oh64 = ((pos // 6) % 64 == jax.lax.broadcasted_iota(jnp.int32, (_BLK, 64), 1)
            ).astype(jnp.float32)
    pe = (jnp.dot(oh6, line_ref[...], preferred_element_type=jnp.float32)
          + jnp.dot(oh2, tri_ref[...], preferred_element_type=jnp.float32)
          + jnp.dot(oh64, hex_ref[...], preferred_element_type=jnp.float32))
    out_ref[...] = scale_ref[0] * (seq_ref[...] + pe)


def kernel(seq_len, line_emb, trigram_emb, hexagram_emb, seq_emb, scale):
    del seq_len
    scale_v = jnp.reshape(scale, (1,))
    return pl.pallas_call(
        _tc_body,
        grid=(_N // _BLK,),
        in_specs=[
            pl.BlockSpec(memory_space=pltpu.SMEM),
            pl.BlockSpec((6, _D), lambda i: (0, 0)),
            pl.BlockSpec((2, _D), lambda i: (0, 0)),
            pl.BlockSpec((64, _D), lambda i: (0, 0)),
            pl.BlockSpec((_BLK, _D), lambda i: (i, 0)),
        ],
        out_specs=pl.BlockSpec((_BLK, _D), lambda i: (i, 0)),
        out_shape=jax.ShapeDtypeStruct((_N, _D), jnp.float32),
    )(scale_v, line_emb, trigram_emb, hexagram_emb, seq_emb)
